# baseline (speedup 1.0000x reference)
import functools

import jax
import jax.numpy as jnp
from jax import lax
from jax.experimental import pallas as pl
from jax.experimental.pallas import tpu as pltpu
from jax.experimental.pallas import tpu_sc as plsc

N_NODES = 10000
N_EDGES = 160000
F_IN = 128
F_OUT = 16
D_EDGE = 16
BN_EPS = 1e-3

NC = 2
NS = 16
NW = NC * NS
C = 128
NBLK = N_EDGES // C
BASE_BLK = NBLK // NW
EXTRA_W = NBLK - BASE_BLK * NW
MAX_BLK = BASE_BLK + 1
HALF_IT = (MAX_BLK + 1) // 2
ROWS_PER_TILE = N_NODES // NS
D_Z = (D_EDGE + 1) * F_OUT


def _precompute_body(x_ref, wa_ref, rk_ref, za_ref, zb_ref, zc_ref, r_ref):
    x = x_ref[...]
    z = jnp.dot(x, wa_ref[...], preferred_element_type=jnp.float32)
    za_ref[...] = z[:, 0:128]
    zb_ref[...] = z[:, 128:256]
    zc_ref[...] = z[:, 256:D_Z]
    r_ref[...] = jnp.dot(x, rk_ref[...], preferred_element_type=jnp.float32)


EAT_BLK = 10


def _ea_transpose_body(ea_ref, eat_ref):
    eat_ref[...] = jnp.transpose(
        ea_ref[...].reshape(EAT_BLK, C, D_EDGE), (0, 2, 1))


def _edge_body(za_hbm, zb_hbm, zc_hbm, ei_hbm, ea_hbm, out_hbm,
               srcall, dstall, zbufa, zbufb, zbufc, eabuf, msgs, ztile, agg_sh,
               gsem0, gsem1, easem0, easem1, ssem0, ssem1):
    cid = lax.axis_index("c")
    sid = lax.axis_index("s")
    wid = sid * NC + cid
    nblk = BASE_BLK + jnp.where(wid < EXTRA_W, 1, 0)
    start = BASE_BLK * wid + jnp.minimum(wid, EXTRA_W)

    gsems = (gsem0, gsem1)
    easems = (easem0, easem1)
    ssems = (ssem0, ssem1)

    pltpu.sync_copy(ei_hbm.at[0, pl.ds(start * C, BASE_BLK * C)],
                    srcall.at[pl.ds(0, BASE_BLK * C)])
    pltpu.sync_copy(ei_hbm.at[1, pl.ds(start * C, BASE_BLK * C)],
                    dstall.at[pl.ds(0, BASE_BLK * C)])

    @pl.when(wid < EXTRA_W)
    def _():
        pltpu.sync_copy(ei_hbm.at[0, pl.ds((start + BASE_BLK) * C, C)],
                        srcall.at[pl.ds(BASE_BLK * C, C)])
        pltpu.sync_copy(ei_hbm.at[1, pl.ds((start + BASE_BLK) * C, C)],
                        dstall.at[pl.ds(BASE_BLK * C, C)])

    zero16 = jnp.zeros((16,), jnp.float32)

    def _zrow(i, carry):
        ztile[i, :] = zero16
        return carry

    lax.fori_loop(0, ROWS_PER_TILE, _zrow, 0)
    pltpu.sync_copy(ztile, agg_sh.at[pl.ds(sid * ROWS_PER_TILE, ROWS_PER_TILE)])
    plsc.subcore_barrier()

    def _fire(i, slot):
        idx = srcall.at[pl.ds(i * C, C)]
        pltpu.async_copy(za_hbm.at[idx], zbufa.at[slot], gsems[slot])
        pltpu.async_copy(zb_hbm.at[idx], zbufb.at[slot], gsems[slot])
        pltpu.async_copy(zc_hbm.at[idx], zbufc.at[slot], gsems[slot])
        pltpu.async_copy(ea_hbm.at[start + i], eabuf.at[slot], easems[slot])

    def _drain(i, slot):
        idx = srcall.at[pl.ds(i * C, C)]
        pltpu.make_async_copy(za_hbm.at[idx], zbufa.at[slot],
                              gsems[slot]).wait()
        pltpu.make_async_copy(zb_hbm.at[idx], zbufb.at[slot],
                              gsems[slot]).wait()
        pltpu.make_async_copy(zc_hbm.at[idx], zbufc.at[slot],
                              gsems[slot]).wait()
        pltpu.make_async_copy(ea_hbm.at[start + i], eabuf.at[slot],
                              easems[slot]).wait()

    def _scat_wait(i, slot):
        pltpu.make_async_copy(msgs.at[slot],
                              agg_sh.at[dstall.at[pl.ds(i * C, C)]],
                              ssems[slot]).wait()

    def _consume(i, slot):
        _drain(i, slot)

        @pl.when(i >= 2)
        def _():
            _scat_wait(i, slot)

        def _group(g, gcarry):
            base = g * 16
            dvec = [eabuf[slot, d, pl.ds(base, 16)] for d in range(D_EDGE)]
            for e16 in range(16):
                e = base + e16
                t = [dvec[d][e16] * zbufa[slot, e, pl.ds(d * F_OUT, F_OUT)]
                     for d in range(8)]
                t += [dvec[d][e16]
                      * zbufb[slot, e, pl.ds((d - 8) * F_OUT, F_OUT)]
                      for d in range(8, D_EDGE)]
                t.append(zbufc[slot, e, :])
                while len(t) > 1:
                    t = [t[i2] + t[i2 + 1]
                         for i2 in range(0, len(t) - 1, 2)] + (
                        [t[-1]] if len(t) % 2 else [])
                msgs[slot, e, :] = t[0]
            return gcarry

        lax.fori_loop(0, C // 16, _group, 0)
        pltpu.async_copy(msgs.at[slot], agg_sh.at[dstall.at[pl.ds(i * C, C)]],
                         ssems[slot], add=True)

    @pl.when(0 < nblk)
    def _():
        _fire(0, 0)

    def _pipe(j, carry):
        b0 = 2 * j
        b1 = 2 * j + 1

        @pl.when(b1 < nblk)
        def _():
            _fire(b1, 1)

        @pl.when(b0 < nblk)
        def _():
            _consume(b0, 0)

        @pl.when(b0 + 2 < nblk)
        def _():
            _fire(b0 + 2, 0)

        @pl.when(b1 < nblk)
        def _():
            _consume(b1, 1)

        return carry

    lax.fori_loop(0, HALF_IT, _pipe, 0)

    _scat_wait(0, 0)
    _scat_wait(0, 1)

    plsc.subcore_barrier()
    pltpu.sync_copy(
        agg_sh.at[pl.ds(sid * ROWS_PER_TILE, ROWS_PER_TILE)],
        out_hbm.at[cid, pl.ds(sid * ROWS_PER_TILE, ROWS_PER_TILE)],
    )


def _epilogue_body(p_ref, r_ref, cb_ref, gamma_ref, beta_ref, mean_ref,
                   var_ref, wd_ref, bd_ref, out_ref):
    half = NC * N_NODES * F_OUT // (2 * 128)
    agg = p_ref[pl.ds(0, half), :] + p_ref[pl.ds(half, half), :]
    agg = agg + r_ref[...] + cb_ref[...]
    out = jnp.maximum(agg, 0.0)
    scale = gamma_ref[...] * lax.rsqrt(var_ref[...] + BN_EPS)
    out = (out - mean_ref[...]) * scale + beta_ref[...]
    pooled128 = jnp.max(out, axis=0, keepdims=True)
    pooled = pooled128[:, 0:F_OUT]
    for k in range(1, 8):
        pooled = jnp.maximum(pooled, pooled128[:, k * F_OUT:(k + 1) * F_OUT])
    out_ref[...] = (
        jnp.dot(pooled, wd_ref[...], preferred_element_type=jnp.float32)
        + bd_ref[...]
    )


def kernel(x, edge_index, edge_attr, Wk, bk, root_kernel, conv_bias,
           gamma, beta, moving_mean, moving_var, Wd, bd):
    wk_r = jnp.transpose(Wk.reshape(D_EDGE, F_IN, F_OUT), (1, 0, 2))
    wa = jnp.concatenate(
        [wk_r.reshape(F_IN, D_EDGE * F_OUT), bk.reshape(F_IN, F_OUT)], axis=1)

    za, zb, zc, r = pl.pallas_call(
        _precompute_body,
        out_shape=[
            jax.ShapeDtypeStruct((N_NODES, 128), jnp.float32),
            jax.ShapeDtypeStruct((N_NODES, 128), jnp.float32),
            jax.ShapeDtypeStruct((N_NODES, F_OUT), jnp.float32),
            jax.ShapeDtypeStruct((N_NODES, F_OUT), jnp.float32),
        ],
    )(x, wa, root_kernel)

    ea_t = pl.pallas_call(
        _ea_transpose_body,
        grid=(NBLK // EAT_BLK,),
        in_specs=[pl.BlockSpec((EAT_BLK * C, D_EDGE), lambda b: (b, 0))],
        out_specs=pl.BlockSpec((EAT_BLK, D_EDGE, C), lambda b: (b, 0, 0)),
        out_shape=jax.ShapeDtypeStruct((NBLK, D_EDGE, C), jnp.float32),
    )(edge_attr)

    mesh = plsc.VectorSubcoreMesh(core_axis_name="c", subcore_axis_name="s")
    edge_fn = functools.partial(
        pl.kernel,
        out_type=jax.ShapeDtypeStruct((NC, N_NODES, F_OUT), jnp.float32),
        mesh=mesh,
        compiler_params=pltpu.CompilerParams(use_tc_tiling_on_sc=False),
        scratch_types=[
            pltpu.VMEM((MAX_BLK * C,), jnp.int32),
            pltpu.VMEM((MAX_BLK * C,), jnp.int32),
            pltpu.VMEM((2, C, 128), jnp.float32),
            pltpu.VMEM((2, C, 128), jnp.float32),
            pltpu.VMEM((2, C, F_OUT), jnp.float32),
            pltpu.VMEM((2, D_EDGE, C), jnp.float32),
            pltpu.VMEM((2, C, F_OUT), jnp.float32),
            pltpu.VMEM((ROWS_PER_TILE, F_OUT), jnp.float32),
            pltpu.VMEM_SHARED((N_NODES, F_OUT), jnp.float32),
            pltpu.SemaphoreType.DMA,
            pltpu.SemaphoreType.DMA,
            pltpu.SemaphoreType.DMA,
            pltpu.SemaphoreType.DMA,
            pltpu.SemaphoreType.DMA,
            pltpu.SemaphoreType.DMA,
        ],
    )(_edge_body)
    partials = edge_fn(za, zb, zc, edge_index, ea_t)

    p128 = partials.reshape(NC * N_NODES * F_OUT // 128, 128)
    r128 = r.reshape(N_NODES * F_OUT // 128, 128)
    tile8 = lambda v: jnp.tile(v.reshape(1, F_OUT), (1, 8))
    logits = pl.pallas_call(
        _epilogue_body,
        out_shape=jax.ShapeDtypeStruct((1, 3), jnp.float32),
    )(
        p128, r128,
        tile8(conv_bias),
        tile8(gamma),
        tile8(beta),
        tile8(moving_mean),
        tile8(moving_var),
        Wd,
        bd.reshape(1, 3),
    )
    return logits


# speedup vs baseline: 1.3753x; 1.3753x over previous
import functools

import jax
import jax.numpy as jnp
from jax import lax
from jax.experimental import pallas as pl
from jax.experimental.pallas import tpu as pltpu
from jax.experimental.pallas import tpu_sc as plsc

N_NODES = 10000
N_EDGES = 160000
F_IN = 128
F_OUT = 16
D_EDGE = 16
BN_EPS = 1e-3

NC = 2
NS = 16
NW = NC * NS
C = 128
NBLK = N_EDGES // C
BASE_BLK = NBLK // NW
EXTRA_W = NBLK - BASE_BLK * NW
MAX_BLK = BASE_BLK + 1
HALF_IT = (MAX_BLK + 1) // 2
ROWS_PER_TILE = N_NODES // NS
D_Z = (D_EDGE + 1) * F_OUT


def _precompute_body(x_ref, wa_ref, rk_ref, za_ref, zb_ref, zc_ref, r_ref):
    x = x_ref[...]
    z = jnp.dot(x, wa_ref[...], preferred_element_type=jnp.float32)
    za_ref[...] = z[:, 0:128]
    zb_ref[...] = z[:, 128:256]
    zc_ref[...] = z[:, 256:D_Z]
    r_ref[...] = jnp.dot(x, rk_ref[...], preferred_element_type=jnp.float32)


def _edge_body(za_hbm, zb_hbm, zc_hbm, ei_hbm, ea_hbm, out_hbm,
               srcall, dstall, zbufa, zbufb, zbufc, eabuf, msgs, ztile, agg_sh,
               gsem0, gsem1, easem0, easem1, ssem0, ssem1):
    cid = lax.axis_index("c")
    sid = lax.axis_index("s")
    wid = sid * NC + cid
    nblk = BASE_BLK + jnp.where(wid < EXTRA_W, 1, 0)
    start = BASE_BLK * wid + jnp.minimum(wid, EXTRA_W)

    gsems = (gsem0, gsem1)
    easems = (easem0, easem1)
    ssems = (ssem0, ssem1)

    pltpu.sync_copy(ei_hbm.at[0, pl.ds(start * C, BASE_BLK * C)],
                    srcall.at[pl.ds(0, BASE_BLK * C)])
    pltpu.sync_copy(ei_hbm.at[1, pl.ds(start * C, BASE_BLK * C)],
                    dstall.at[pl.ds(0, BASE_BLK * C)])

    @pl.when(wid < EXTRA_W)
    def _():
        pltpu.sync_copy(ei_hbm.at[0, pl.ds((start + BASE_BLK) * C, C)],
                        srcall.at[pl.ds(BASE_BLK * C, C)])
        pltpu.sync_copy(ei_hbm.at[1, pl.ds((start + BASE_BLK) * C, C)],
                        dstall.at[pl.ds(BASE_BLK * C, C)])

    zero16 = jnp.zeros((16,), jnp.float32)

    def _zrow(i, carry):
        ztile[i, :] = zero16
        return carry

    lax.fori_loop(0, ROWS_PER_TILE, _zrow, 0)
    pltpu.sync_copy(ztile, agg_sh.at[pl.ds(sid * ROWS_PER_TILE, ROWS_PER_TILE)])
    plsc.subcore_barrier()

    def _fire(i, slot):
        idx = srcall.at[pl.ds(i * C, C)]
        pltpu.async_copy(za_hbm.at[idx], zbufa.at[slot], gsems[slot])
        pltpu.async_copy(zb_hbm.at[idx], zbufb.at[slot], gsems[slot])
        pltpu.async_copy(zc_hbm.at[idx], zbufc.at[slot], gsems[slot])
        pltpu.async_copy(ea_hbm.at[pl.ds((start + i) * C, C)],
                         eabuf.at[slot], easems[slot])

    def _drain(i, slot):
        idx = srcall.at[pl.ds(i * C, C)]
        pltpu.make_async_copy(za_hbm.at[idx], zbufa.at[slot],
                              gsems[slot]).wait()
        pltpu.make_async_copy(zb_hbm.at[idx], zbufb.at[slot],
                              gsems[slot]).wait()
        pltpu.make_async_copy(zc_hbm.at[idx], zbufc.at[slot],
                              gsems[slot]).wait()
        pltpu.make_async_copy(ea_hbm.at[pl.ds((start + i) * C, C)],
                              eabuf.at[slot], easems[slot]).wait()

    def _scat_wait(i, slot):
        pltpu.make_async_copy(msgs.at[slot],
                              agg_sh.at[dstall.at[pl.ds(i * C, C)]],
                              ssems[slot]).wait()

    def _consume(i, slot):
        _drain(i, slot)

        @pl.when(i >= 2)
        def _():
            _scat_wait(i, slot)

        def _edge(e, ecarry):
            ea_row = eabuf[slot, e, :]
            t = [ea_row[d] * zbufa[slot, e, pl.ds(d * F_OUT, F_OUT)]
                 for d in range(8)]
            t += [ea_row[d] * zbufb[slot, e, pl.ds((d - 8) * F_OUT, F_OUT)]
                  for d in range(8, D_EDGE)]
            t.append(zbufc[slot, e, :])
            while len(t) > 1:
                t = [t[i2] + t[i2 + 1] for i2 in range(0, len(t) - 1, 2)] + (
                    [t[-1]] if len(t) % 2 else [])
            msgs[slot, e, :] = t[0]
            return ecarry

        lax.fori_loop(0, C, _edge, 0, unroll=4)
        pltpu.async_copy(msgs.at[slot], agg_sh.at[dstall.at[pl.ds(i * C, C)]],
                         ssems[slot], add=True)

    @pl.when(0 < nblk)
    def _():
        _fire(0, 0)

    def _pipe(j, carry):
        b0 = 2 * j
        b1 = 2 * j + 1

        @pl.when(b1 < nblk)
        def _():
            _fire(b1, 1)

        @pl.when(b0 < nblk)
        def _():
            _consume(b0, 0)

        @pl.when(b0 + 2 < nblk)
        def _():
            _fire(b0 + 2, 0)

        @pl.when(b1 < nblk)
        def _():
            _consume(b1, 1)

        return carry

    lax.fori_loop(0, HALF_IT, _pipe, 0)

    _scat_wait(0, 0)
    _scat_wait(0, 1)

    plsc.subcore_barrier()
    pltpu.sync_copy(
        agg_sh.at[pl.ds(sid * ROWS_PER_TILE, ROWS_PER_TILE)],
        out_hbm.at[cid, pl.ds(sid * ROWS_PER_TILE, ROWS_PER_TILE)],
    )


def _epilogue_body(p_ref, r_ref, cb_ref, gamma_ref, beta_ref, mean_ref,
                   var_ref, wd_ref, bd_ref, out_ref):
    half = NC * N_NODES * F_OUT // (2 * 128)
    agg = p_ref[pl.ds(0, half), :] + p_ref[pl.ds(half, half), :]
    agg = agg + r_ref[...] + cb_ref[...]
    out = jnp.maximum(agg, 0.0)
    scale = gamma_ref[...] * lax.rsqrt(var_ref[...] + BN_EPS)
    out = (out - mean_ref[...]) * scale + beta_ref[...]
    pooled128 = jnp.max(out, axis=0, keepdims=True)
    pooled = pooled128[:, 0:F_OUT]
    for k in range(1, 8):
        pooled = jnp.maximum(pooled, pooled128[:, k * F_OUT:(k + 1) * F_OUT])
    out_ref[...] = (
        jnp.dot(pooled, wd_ref[...], preferred_element_type=jnp.float32)
        + bd_ref[...]
    )


def kernel(x, edge_index, edge_attr, Wk, bk, root_kernel, conv_bias,
           gamma, beta, moving_mean, moving_var, Wd, bd):
    wk_r = jnp.transpose(Wk.reshape(D_EDGE, F_IN, F_OUT), (1, 0, 2))
    wa = jnp.concatenate(
        [wk_r.reshape(F_IN, D_EDGE * F_OUT), bk.reshape(F_IN, F_OUT)], axis=1)

    za, zb, zc, r = pl.pallas_call(
        _precompute_body,
        out_shape=[
            jax.ShapeDtypeStruct((N_NODES, 128), jnp.float32),
            jax.ShapeDtypeStruct((N_NODES, 128), jnp.float32),
            jax.ShapeDtypeStruct((N_NODES, F_OUT), jnp.float32),
            jax.ShapeDtypeStruct((N_NODES, F_OUT), jnp.float32),
        ],
    )(x, wa, root_kernel)

    mesh = plsc.VectorSubcoreMesh(core_axis_name="c", subcore_axis_name="s")
    edge_fn = functools.partial(
        pl.kernel,
        out_type=jax.ShapeDtypeStruct((NC, N_NODES, F_OUT), jnp.float32),
        mesh=mesh,
        compiler_params=pltpu.CompilerParams(use_tc_tiling_on_sc=False),
        scratch_types=[
            pltpu.VMEM((MAX_BLK * C,), jnp.int32),
            pltpu.VMEM((MAX_BLK * C,), jnp.int32),
            pltpu.VMEM((2, C, 128), jnp.float32),
            pltpu.VMEM((2, C, 128), jnp.float32),
            pltpu.VMEM((2, C, F_OUT), jnp.float32),
            pltpu.VMEM((2, C, D_EDGE), jnp.float32),
            pltpu.VMEM((2, C, F_OUT), jnp.float32),
            pltpu.VMEM((ROWS_PER_TILE, F_OUT), jnp.float32),
            pltpu.VMEM_SHARED((N_NODES, F_OUT), jnp.float32),
            pltpu.SemaphoreType.DMA,
            pltpu.SemaphoreType.DMA,
            pltpu.SemaphoreType.DMA,
            pltpu.SemaphoreType.DMA,
            pltpu.SemaphoreType.DMA,
            pltpu.SemaphoreType.DMA,
        ],
    )(_edge_body)
    partials = edge_fn(za, zb, zc, edge_index, edge_attr)

    p128 = partials.reshape(NC * N_NODES * F_OUT // 128, 128)
    r128 = r.reshape(N_NODES * F_OUT // 128, 128)
    tile8 = lambda v: jnp.tile(v.reshape(1, F_OUT), (1, 8))
    logits = pl.pallas_call(
        _epilogue_body,
        out_shape=jax.ShapeDtypeStruct((1, 3), jnp.float32),
    )(
        p128, r128,
        tile8(conv_bias),
        tile8(gamma),
        tile8(beta),
        tile8(moving_mean),
        tile8(moving_var),
        Wd,
        bd.reshape(1, 3),
    )
    return logits
